# Initial kernel scaffold; baseline (speedup 1.0000x reference)
import jax
import jax.numpy as jnp
from jax.experimental import pallas as pl


def kernel(encoder_hidden, decoder_hidden, relative_attention_bias):
    raise NotImplementedError("write your pallas kernel here")


# speedup vs baseline: 42.3473x; 42.3473x over previous
import jax
import jax.numpy as jnp
from jax import lax
from jax.experimental import pallas as pl
from jax.experimental.pallas import tpu as pltpu
import jax.experimental.pallas.tpu_sc as plsc

Q = 2048
K = 2048
H = 16
NSHIFT = 8
TWC = 257
TW = TWC * 16
NCHUNK = 192
ROWS_PER_SUB = Q // 2

_THRESH = (12, 16, 23, 32, 46, 64, 91)


def _sc_body(bias_hbm, out_hbm, bias_v, tbl_v, sem):
    c = lax.axis_index("c")
    s = lax.axis_index("s")
    h = s
    hb = c

    pltpu.sync_copy(bias_hbm, bias_v)

    lane = lax.iota(jnp.int32, 16)
    hvec = jnp.full((16,), h, jnp.int32)

    chunk_lo = (1 - hb) * 64

    for u in range(NSHIFT):
        dshift = (7 - u) - (Q - 1)

        def build(t, carry, _dshift=dshift, _u=u):
            ch = chunk_lo + t
            d = ch * 16 + _dshift + lane
            pos = d > 0
            a = jnp.abs(d)
            large = jnp.full((16,), 8, jnp.int32)
            for idx, thr in enumerate(_THRESH):
                large = jnp.where(a >= thr, 9 + idx, large)
            sub = jnp.where(a < 8, a, large)
            bkt = jnp.where(pos, sub + 16, sub)
            vals = plsc.load_gather(bias_v, [bkt, hvec])
            tbl_v[pl.ds((_u * TWC + ch) * 16, 16)] = vals
            return carry

        lax.fori_loop(0, NCHUNK, build, 0)

    row0 = hb * ROWS_PER_SUB

    def fire(r, carry):
        i = row0 + r
        o = (Q - 1) - i
        sft = jnp.bitwise_and(o, 7)
        src_off8 = (7 - sft) * (TW // 8) + jnp.right_shift(o, 3)
        pltpu.async_copy(
            tbl_v.at[pl.ds(src_off8 * 8, K)],
            out_hbm.at[pl.ds((h * Q + i) * K, K)],
            sem,
        )
        return carry

    lax.fori_loop(0, ROWS_PER_SUB, fire, 0)

    def drain(r, carry):
        pltpu.make_async_copy(
            out_hbm.at[pl.ds(0, K)], tbl_v.at[pl.ds(0, K)], sem
        ).wait()
        return carry

    lax.fori_loop(0, ROWS_PER_SUB, drain, 0)


@jax.jit
def _sc_bias(relative_attention_bias):
    mesh = plsc.VectorSubcoreMesh(
        core_axis_name="c", subcore_axis_name="s", num_cores=2, num_subcores=16
    )
    out = pl.kernel(
        _sc_body,
        out_type=jax.ShapeDtypeStruct((H * Q * K,), jnp.float32),
        mesh=mesh,
        scratch_types=[
            pltpu.VMEM((32, H), jnp.float32),
            pltpu.VMEM((NSHIFT * TW,), jnp.float32),
            pltpu.SemaphoreType.DMA,
        ],
        compiler_params=pltpu.CompilerParams(needs_layout_passes=False),
    )(relative_attention_bias)
    return out


def kernel(encoder_hidden, decoder_hidden, relative_attention_bias):
    out = _sc_bias(relative_attention_bias)
    return out.reshape(1, H, Q, K)
